# Initial kernel scaffold; baseline (speedup 1.0000x reference)
import jax
import jax.numpy as jnp
from jax.experimental import pallas as pl


def kernel(x, edge_index, edge_attr, W_edge, b_edge, W1, b1, W2, b2, gamma, beta):
    raise NotImplementedError("write your pallas kernel here")


# speedup vs baseline: 2.4260x; 2.4260x over previous
import functools

import jax
import jax.numpy as jnp
from jax import lax
from jax.experimental import pallas as pl
from jax.experimental.pallas import tpu as pltpu
from jax.experimental.pallas import tpu_sc as plsc

N = 10000
E = 320000
D = 128
ED = 10

NC = 2
NS = 16
NW = NC * NS
EPW = E // NW
CH = 80
NCHUNK = EPW // CH
NP = 10240
ROWS_PT = NP // NS


EB = 2560


def _edge_lin_body(attr_ref, w_ref, b_ref, out_ref):
    out_ref[...] = (
        jnp.dot(attr_ref[...], w_ref[...], preferred_element_type=jnp.float32)
        + b_ref[...]
    )


def _edge_lin(edge_attr, W_edge, b_edge):
    return pl.pallas_call(
        _edge_lin_body,
        grid=(E // EB,),
        in_specs=[
            pl.BlockSpec((EB, ED), lambda i: (i, 0)),
            pl.BlockSpec((ED, D), lambda i: (0, 0)),
            pl.BlockSpec((1, D), lambda i: (0, 0)),
        ],
        out_specs=pl.BlockSpec((EB, D), lambda i: (i, 0)),
        out_shape=jax.ShapeDtypeStruct((E, D), jnp.float32),
    )(edge_attr, W_edge, b_edge.reshape(1, D))


def _sc_edge_body(src_hbm, dst_hbm, e_hbm, x_hbm, zeros_hbm, out_hbm,
                  src_v, dst_v, rows_v, e_v, aggr_sh, sem):
    cid = lax.axis_index("c")
    sid = lax.axis_index("s")
    wid = sid * NC + cid

    r0 = sid * ROWS_PT
    pltpu.sync_copy(zeros_hbm.at[pl.ds(r0, ROWS_PT)],
                    aggr_sh.at[pl.ds(r0, ROWS_PT)])
    plsc.subcore_barrier()

    base = wid * EPW

    def chunk(c, carry):
        off = base + c * CH
        pltpu.sync_copy(src_hbm.at[pl.ds(off, CH)], src_v)
        pltpu.sync_copy(dst_hbm.at[pl.ds(off, CH)], dst_v)
        pltpu.async_copy(x_hbm.at[src_v], rows_v, sem).wait()
        pltpu.sync_copy(e_hbm.at[pl.ds(off, CH)], e_v)

        def row(i, carry2):
            for j in range(D // 16):
                sl = pl.ds(j * 16, 16)
                rows_v[i, sl] = jnp.maximum(rows_v[i, sl] + e_v[i, sl], 0.0)
            return carry2

        lax.fori_loop(0, CH, row, 0)
        pltpu.sync_copy(rows_v, aggr_sh.at[dst_v], add=True)
        return carry

    lax.fori_loop(0, NCHUNK, chunk, 0)

    plsc.subcore_barrier()
    pltpu.sync_copy(aggr_sh.at[pl.ds(r0, ROWS_PT)],
                    out_hbm.at[cid, pl.ds(r0, ROWS_PT)])


def _sc_edge(src, dst, e, x, zeros):
    mesh = plsc.VectorSubcoreMesh(core_axis_name="c", subcore_axis_name="s")
    call = pl.kernel(
        _sc_edge_body,
        out_type=jax.ShapeDtypeStruct((NC, NP, D), jnp.float32),
        mesh=mesh,
        scratch_types=[
            pltpu.VMEM((CH,), jnp.int32),
            pltpu.VMEM((CH,), jnp.int32),
            pltpu.VMEM((CH, D), jnp.float32),
            pltpu.VMEM((CH, D), jnp.float32),
            pltpu.VMEM_SHARED((NP, D), jnp.float32),
            pltpu.SemaphoreType.DMA,
        ],
    )
    return call(src, dst, e, x, zeros)


NB = 2000


def _mlp_body(x_ref, a0_ref, a1_ref, w1_ref, b1_ref, w2_ref, b2_ref,
              h_ref, s_ref, ss_ref):
    out = x_ref[...] + a0_ref[0] + a1_ref[0]
    h = jnp.maximum(
        jnp.dot(out, w1_ref[...], preferred_element_type=jnp.float32)
        + b1_ref[...], 0.0)
    h = jnp.maximum(
        jnp.dot(h, w2_ref[...], preferred_element_type=jnp.float32)
        + b2_ref[...], 0.0)
    h_ref[...] = h

    @pl.when(pl.program_id(0) == 0)
    def _():
        s_ref[...] = jnp.zeros_like(s_ref)
        ss_ref[...] = jnp.zeros_like(ss_ref)

    s_ref[...] += jnp.sum(h, axis=0, keepdims=True)
    ss_ref[...] += jnp.sum(h * h, axis=0, keepdims=True)


def _node_mlp(x, partials, W1, b1, W2, b2):
    return pl.pallas_call(
        _mlp_body,
        grid=(N // NB,),
        in_specs=[
            pl.BlockSpec((NB, D), lambda i: (i, 0)),
            pl.BlockSpec((1, NB, D), lambda i: (0, i, 0)),
            pl.BlockSpec((1, NB, D), lambda i: (1, i, 0)),
            pl.BlockSpec((D, D), lambda i: (0, 0)),
            pl.BlockSpec((1, D), lambda i: (0, 0)),
            pl.BlockSpec((D, D), lambda i: (0, 0)),
            pl.BlockSpec((1, D), lambda i: (0, 0)),
        ],
        out_specs=[
            pl.BlockSpec((NB, D), lambda i: (i, 0)),
            pl.BlockSpec((1, D), lambda i: (0, 0)),
            pl.BlockSpec((1, D), lambda i: (0, 0)),
        ],
        out_shape=[
            jax.ShapeDtypeStruct((N, D), jnp.float32),
            jax.ShapeDtypeStruct((1, D), jnp.float32),
            jax.ShapeDtypeStruct((1, D), jnp.float32),
        ],
    )(x, partials, partials, W1, b1.reshape(1, D), W2, b2.reshape(1, D))


def _bn_body(h_ref, s_ref, ss_ref, g_ref, bt_ref, o_ref):
    mean = s_ref[...] * (1.0 / N)
    var = ss_ref[...] * (1.0 / N) - mean * mean
    inv = lax.rsqrt(var + 1e-5)
    o_ref[...] = (h_ref[...] - mean) * inv * g_ref[...] + bt_ref[...]


def _bn(h, s, ss, gamma, beta):
    return pl.pallas_call(
        _bn_body,
        grid=(N // NB,),
        in_specs=[
            pl.BlockSpec((NB, D), lambda i: (i, 0)),
            pl.BlockSpec((1, D), lambda i: (0, 0)),
            pl.BlockSpec((1, D), lambda i: (0, 0)),
            pl.BlockSpec((1, D), lambda i: (0, 0)),
            pl.BlockSpec((1, D), lambda i: (0, 0)),
        ],
        out_specs=pl.BlockSpec((NB, D), lambda i: (i, 0)),
        out_shape=jax.ShapeDtypeStruct((N, D), jnp.float32),
    )(h, s, ss, gamma.reshape(1, D), beta.reshape(1, D))


def kernel(x, edge_index, edge_attr, W_edge, b_edge, W1, b1, W2, b2, gamma,
           beta):
    src = edge_index[0]
    dst = edge_index[1]
    e = _edge_lin(edge_attr, W_edge, b_edge)
    zeros = jnp.zeros((NP, D), jnp.float32)
    partials = _sc_edge(src, dst, e, x, zeros)
    h, s, ss = _node_mlp(x, partials, W1, b1, W2, b2)
    return _bn(h, s, ss, gamma, beta)


# speedup vs baseline: 3.8092x; 1.5701x over previous
import functools

import jax
import jax.numpy as jnp
from jax import lax
from jax.experimental import pallas as pl
from jax.experimental.pallas import tpu as pltpu
from jax.experimental.pallas import tpu_sc as plsc

N = 10000
E = 320000
D = 128
ED = 10

NC = 2
NS = 16
NW = NC * NS
EPW = E // NW
CH = 80
NCHUNK = EPW // CH
NP = 10240
ROWS_PT = NP // NS


EB = 2560


def _edge_lin_body(attr_ref, w_ref, b_ref, out_ref):
    out_ref[...] = (
        jnp.dot(attr_ref[...], w_ref[...], preferred_element_type=jnp.float32)
        + b_ref[...]
    )


def _edge_lin(edge_attr, W_edge, b_edge):
    return pl.pallas_call(
        _edge_lin_body,
        grid=(E // EB,),
        in_specs=[
            pl.BlockSpec((EB, ED), lambda i: (i, 0)),
            pl.BlockSpec((ED, D), lambda i: (0, 0)),
            pl.BlockSpec((1, D), lambda i: (0, 0)),
        ],
        out_specs=pl.BlockSpec((EB, D), lambda i: (i, 0)),
        out_shape=jax.ShapeDtypeStruct((E, D), jnp.float32),
    )(edge_attr, W_edge, b_edge.reshape(1, D))


def _sc_edge_body(src_hbm, dst_hbm, e_hbm, x_hbm, zeros_hbm, out_hbm,
                  src_v, dst_v, dst_s, rows_v, e_v, aggr_sh,
                  sem_i, sem_g, sem_e, sem_s):
    cid = lax.axis_index("c")
    sid = lax.axis_index("s")
    wid = sid * NC + cid

    r0 = sid * ROWS_PT
    pltpu.sync_copy(zeros_hbm.at[pl.ds(r0, ROWS_PT)],
                    aggr_sh.at[pl.ds(r0, ROWS_PT)])
    plsc.subcore_barrier()

    base = wid * EPW

    def off(c):
        return base + jnp.minimum(c, NCHUNK - 1) * CH

    def issue_idx(b, c):
        o = off(c)
        pltpu.async_copy(src_hbm.at[pl.ds(o, CH)], src_v[b], sem_i[b])
        pltpu.async_copy(dst_hbm.at[pl.ds(o, CH)], dst_v[b], sem_i[b])

    def wait_idx(b):
        pltpu.make_async_copy(src_hbm.at[pl.ds(base, CH)], src_v[b],
                              sem_i[b]).wait()
        pltpu.make_async_copy(dst_hbm.at[pl.ds(base, CH)], dst_v[b],
                              sem_i[b]).wait()

    def issue_ge(b, c):
        pltpu.async_copy(x_hbm.at[src_v[b]], rows_v[b], sem_g[b])
        pltpu.async_copy(e_hbm.at[pl.ds(off(c), CH)], e_v[b], sem_e[b])

    def wait_ge(b):
        pltpu.make_async_copy(x_hbm.at[src_v[b]], rows_v[b], sem_g[b]).wait()
        pltpu.make_async_copy(e_hbm.at[pl.ds(base, CH)], e_v[b],
                              sem_e[b]).wait()

    def copy_dst(b):
        for k in range(CH // 16):
            sl = pl.ds(k * 16, 16)
            dst_s[b][sl] = dst_v[b][sl]

    def compute(b):
        def row4(k, carry):
            i = k * 4
            for r in range(4):
                for j in range(D // 16):
                    sl = pl.ds(j * 16, 16)
                    rows_v[b][i + r, sl] = jnp.maximum(
                        rows_v[b][i + r, sl] + e_v[b][i + r, sl], 0.0)
            return carry

        lax.fori_loop(0, CH // 4, row4, 0)

    def issue_s(b):
        pltpu.async_copy(rows_v[b], aggr_sh.at[dst_s[b]], sem_s[b], add=True)

    def wait_s(b):
        pltpu.make_async_copy(rows_v[b], aggr_sh.at[dst_s[b]],
                              sem_s[b]).wait()

    issue_idx(0, 0)
    issue_idx(1, 1)
    wait_idx(0)
    issue_ge(0, 0)
    wait_idx(1)
    issue_ge(1, 1)
    wait_ge(0)
    copy_dst(0)
    issue_idx(0, 2)
    compute(0)
    issue_s(0)
    wait_ge(1)
    copy_dst(1)
    issue_idx(1, 3)
    compute(1)
    issue_s(1)
    wait_s(0)
    wait_idx(0)
    issue_ge(0, 2)

    def pair(it, carry):
        c = 2 + 2 * it
        wait_s(1)
        wait_idx(1)
        issue_ge(1, c + 1)
        wait_ge(0)
        copy_dst(0)
        issue_idx(0, c + 2)
        compute(0)
        issue_s(0)
        wait_ge(1)
        copy_dst(1)
        issue_idx(1, c + 3)
        compute(1)
        issue_s(1)
        wait_s(0)
        wait_idx(0)
        issue_ge(0, c + 2)
        return carry

    lax.fori_loop(0, (NCHUNK - 3) // 2, pair, 0)

    wait_ge(0)
    copy_dst(0)
    compute(0)
    issue_s(0)
    wait_s(1)
    wait_idx(1)
    wait_s(0)

    plsc.subcore_barrier()
    pltpu.sync_copy(aggr_sh.at[pl.ds(r0, ROWS_PT)],
                    out_hbm.at[cid, pl.ds(r0, ROWS_PT)])


def _sc_edge(src, dst, e, x, zeros):
    mesh = plsc.VectorSubcoreMesh(core_axis_name="c", subcore_axis_name="s")
    call = pl.kernel(
        _sc_edge_body,
        out_type=jax.ShapeDtypeStruct((NC, NP, D), jnp.float32),
        mesh=mesh,
        scratch_types=[
            [pltpu.VMEM((CH,), jnp.int32) for _ in range(2)],
            [pltpu.VMEM((CH,), jnp.int32) for _ in range(2)],
            [pltpu.VMEM((CH,), jnp.int32) for _ in range(2)],
            [pltpu.VMEM((CH, D), jnp.float32) for _ in range(2)],
            [pltpu.VMEM((CH, D), jnp.float32) for _ in range(2)],
            pltpu.VMEM_SHARED((NP, D), jnp.float32),
            [pltpu.SemaphoreType.DMA for _ in range(2)],
            [pltpu.SemaphoreType.DMA for _ in range(2)],
            [pltpu.SemaphoreType.DMA for _ in range(2)],
            [pltpu.SemaphoreType.DMA for _ in range(2)],
        ],
    )
    return call(src, dst, e, x, zeros)


NB = 2000


def _mlp_body(x_ref, a0_ref, a1_ref, w1_ref, b1_ref, w2_ref, b2_ref,
              h_ref, s_ref, ss_ref):
    out = x_ref[...] + a0_ref[0] + a1_ref[0]
    h = jnp.maximum(
        jnp.dot(out, w1_ref[...], preferred_element_type=jnp.float32)
        + b1_ref[...], 0.0)
    h = jnp.maximum(
        jnp.dot(h, w2_ref[...], preferred_element_type=jnp.float32)
        + b2_ref[...], 0.0)
    h_ref[...] = h

    @pl.when(pl.program_id(0) == 0)
    def _():
        s_ref[...] = jnp.zeros_like(s_ref)
        ss_ref[...] = jnp.zeros_like(ss_ref)

    s_ref[...] += jnp.sum(h, axis=0, keepdims=True)
    ss_ref[...] += jnp.sum(h * h, axis=0, keepdims=True)


def _node_mlp(x, partials, W1, b1, W2, b2):
    return pl.pallas_call(
        _mlp_body,
        grid=(N // NB,),
        in_specs=[
            pl.BlockSpec((NB, D), lambda i: (i, 0)),
            pl.BlockSpec((1, NB, D), lambda i: (0, i, 0)),
            pl.BlockSpec((1, NB, D), lambda i: (1, i, 0)),
            pl.BlockSpec((D, D), lambda i: (0, 0)),
            pl.BlockSpec((1, D), lambda i: (0, 0)),
            pl.BlockSpec((D, D), lambda i: (0, 0)),
            pl.BlockSpec((1, D), lambda i: (0, 0)),
        ],
        out_specs=[
            pl.BlockSpec((NB, D), lambda i: (i, 0)),
            pl.BlockSpec((1, D), lambda i: (0, 0)),
            pl.BlockSpec((1, D), lambda i: (0, 0)),
        ],
        out_shape=[
            jax.ShapeDtypeStruct((N, D), jnp.float32),
            jax.ShapeDtypeStruct((1, D), jnp.float32),
            jax.ShapeDtypeStruct((1, D), jnp.float32),
        ],
    )(x, partials, partials, W1, b1.reshape(1, D), W2, b2.reshape(1, D))


def _bn_body(h_ref, s_ref, ss_ref, g_ref, bt_ref, o_ref):
    mean = s_ref[...] * (1.0 / N)
    var = ss_ref[...] * (1.0 / N) - mean * mean
    inv = lax.rsqrt(var + 1e-5)
    o_ref[...] = (h_ref[...] - mean) * inv * g_ref[...] + bt_ref[...]


def _bn(h, s, ss, gamma, beta):
    return pl.pallas_call(
        _bn_body,
        grid=(N // NB,),
        in_specs=[
            pl.BlockSpec((NB, D), lambda i: (i, 0)),
            pl.BlockSpec((1, D), lambda i: (0, 0)),
            pl.BlockSpec((1, D), lambda i: (0, 0)),
            pl.BlockSpec((1, D), lambda i: (0, 0)),
            pl.BlockSpec((1, D), lambda i: (0, 0)),
        ],
        out_specs=pl.BlockSpec((NB, D), lambda i: (i, 0)),
        out_shape=jax.ShapeDtypeStruct((N, D), jnp.float32),
    )(h, s, ss, gamma.reshape(1, D), beta.reshape(1, D))


def kernel(x, edge_index, edge_attr, W_edge, b_edge, W1, b1, W2, b2, gamma,
           beta):
    src = edge_index[0]
    dst = edge_index[1]
    e = _edge_lin(edge_attr, W_edge, b_edge)
    zeros = jnp.zeros((NP, D), jnp.float32)
    partials = _sc_edge(src, dst, e, x, zeros)
    h, s, ss = _node_mlp(x, partials, W1, b1, W2, b2)
    return _bn(h, s, ss, gamma, beta)
